# baseline (speedup 1.0000x reference)
import functools

import jax
import jax.numpy as jnp
from jax import lax
from jax.experimental import pallas as pl
from jax.experimental.pallas import tpu as pltpu
from jax.experimental.pallas import tpu_sc as plsc

N_NODES = 10000
C = 128
N_EDGES = 320000

NC = 2
NS = 16
NW = NC * NS

K_CHUNK = 80
N_PHASE = 1
E_HALF = N_EDGES // N_PHASE
N_CHUNKS = E_HALF // K_CHUNK
CHUNKS_PER_W = -(-N_CHUNKS // NW)

ZROWS = 80
N_ZCHUNKS = N_NODES // ZROWS
ZCHUNKS_PER_TILE = -(-N_ZCHUNKS // NS)


_BE = 16000


_DN = (((0,), (0,)), ((), ()))


def _edge_weights_body(rt_ref, yt_ref, wr_ref, wy_ref, o_ref):
    kr = lax.dot_general(rt_ref[...], wr_ref[...], _DN,
                         preferred_element_type=jnp.float32)
    ky = lax.dot_general(yt_ref[...], wy_ref[...], _DN,
                         preferred_element_type=jnp.float32)
    o_ref[...] = kr * ky


def _edge_weights(RT, YT, W_R, W_Y, half):
    n_radial = RT.shape[0]
    n_sh = YT.shape[0]
    hb = E_HALF // _BE
    return pl.pallas_call(
        _edge_weights_body,
        grid=(hb,),
        in_specs=[
            pl.BlockSpec((n_radial, _BE), lambda i: (0, half * hb + i)),
            pl.BlockSpec((n_sh, _BE), lambda i: (0, half * hb + i)),
            pl.BlockSpec((n_radial, C), lambda i: (0, 0)),
            pl.BlockSpec((n_sh, C), lambda i: (0, 0)),
        ],
        out_specs=pl.BlockSpec((_BE, C), lambda i: (i, 0)),
        out_shape=jax.ShapeDtypeStruct((E_HALF, C), jnp.float32),
    )(RT, YT, W_R, W_Y)


_sc_mesh = plsc.VectorSubcoreMesh(
    core_axis_name="c", subcore_axis_name="s", num_cores=NC, num_subcores=NS
)


@functools.partial(
    pl.kernel,
    out_type=jax.ShapeDtypeStruct((NC, N_NODES, C), jnp.float32),
    mesh=_sc_mesh,
    scratch_types=[
        pltpu.VMEM((2, K_CHUNK), jnp.int32),
        pltpu.VMEM((2, K_CHUNK), jnp.int32),
        pltpu.VMEM((2, K_CHUNK), jnp.int32),
        pltpu.VMEM((2, K_CHUNK, C), jnp.float32),
        pltpu.VMEM((2, K_CHUNK, C), jnp.float32),
        pltpu.VMEM_SHARED((N_NODES, C), jnp.float32),
        pltpu.SemaphoreType.DMA,
        pltpu.SemaphoreType.DMA,
        pltpu.SemaphoreType.DMA,
        pltpu.SemaphoreType.DMA,
        pltpu.SemaphoreType.DMA,
        pltpu.SemaphoreType.DMA,
        pltpu.SemaphoreType.DMA,
        pltpu.SemaphoreType.DMA,
    ],
)
def _sc_gather_scatter(feat_hbm, ker_hbm, ia_hbm, ib_hbm, out_hbm,
                       ib_v, ia_v, ia_sc, feat_v, ker_v, acc,
                       si0, si1, sg0, sg1, sk0, sk1, ss0, ss1):
    cid = lax.axis_index("c")
    sid = lax.axis_index("s")
    wid = cid * NS + sid

    s_idx = (si0, si1)
    s_gat = (sg0, sg1)
    s_ker = (sk0, sk1)
    s_sct = (ss0, ss1)

    def d_ib(b, chunk):
        return pltpu.make_async_copy(
            ib_hbm.at[pl.ds(chunk * K_CHUNK, K_CHUNK)], ib_v.at[b], s_idx[b])

    def d_ia(b, chunk):
        return pltpu.make_async_copy(
            ia_hbm.at[pl.ds(chunk * K_CHUNK, K_CHUNK)], ia_v.at[b], s_idx[b])

    def d_gat(b):
        return pltpu.make_async_copy(feat_hbm.at[ib_v.at[b]], feat_v.at[b],
                                     s_gat[b])

    def d_ker(b, chunk):
        return pltpu.make_async_copy(
            ker_hbm.at[pl.ds(chunk * K_CHUNK, K_CHUNK)], ker_v.at[b], s_ker[b])

    def d_sct(b):
        return pltpu.make_async_copy(ker_v.at[b], acc.at[ia_sc.at[b]],
                                     s_sct[b])

    zero16 = jnp.zeros((16,), jnp.float32)

    def _zrow(r, carry):
        for c8 in range(C // 16):
            feat_v[0, r, pl.ds(c8 * 16, 16)] = zero16
        return carry

    lax.fori_loop(0, ZROWS, _zrow, 0)

    def _zchunk(j, carry):
        zc = sid + j * NS

        @pl.when(zc < N_ZCHUNKS)
        def _():
            pltpu.sync_copy(
                feat_v.at[0, pl.ds(0, ZROWS)],
                acc.at[pl.ds(zc * ZROWS, ZROWS)],
            )

        return carry

    lax.fori_loop(0, ZCHUNKS_PER_TILE, _zchunk, 0)
    plsc.subcore_barrier()

    c0 = wid
    d_ib(0, c0).start()
    d_ia(0, c0).start()
    d_ib(0, c0).wait()
    d_ia(0, c0).wait()
    d_gat(0).start()
    d_ker(0, c0).start()

    @pl.when(c0 + NW < N_CHUNKS)
    def _():
        d_ib(1, c0 + NW).start()
        d_ia(1, c0 + NW).start()

    def _step(i, s):
        o = 1 - s
        ci = wid + i * NW

        @pl.when(ci < N_CHUNKS)
        def _():
            cn = ci + NW
            cn2 = cn + NW
            d_gat(s).wait()
            d_ker(s, ci).wait()
            for g in range(K_CHUNK // 16):
                ia_sc[s, pl.ds(g * 16, 16)] = ia_v[s, pl.ds(g * 16, 16)]

            @pl.when(cn < N_CHUNKS)
            def _():
                d_ib(o, cn).wait()
                d_ia(o, cn).wait()

                @pl.when(i >= 1)
                def _():
                    d_sct(o).wait()

                d_gat(o).start()
                d_ker(o, cn).start()

            @pl.when(cn2 < N_CHUNKS)
            def _():
                d_ib(s, cn2).start()
                d_ia(s, cn2).start()

            def _row(r):
                for c8 in range(C // 16):
                    sl = pl.ds(c8 * 16, 16)
                    ker_v[s, r, sl] = ker_v[s, r, sl] * feat_v[s, r, sl]

            plsc.parallel_loop(0, K_CHUNK, unroll=4)(_row)
            d_sct(s).start(add=True)

    def _pair(it, carry):
        _step(it * 2, 0)
        _step(it * 2 + 1, 1)
        return carry

    lax.fori_loop(0, (CHUNKS_PER_W + 1) // 2, _pair, 0)

    lv = (N_CHUNKS - 1 - wid) // NW

    @pl.when(lv % 2 == 0)
    def _():
        @pl.when(lv >= 1)
        def _():
            d_sct(1).wait()

        d_sct(0).wait()

    @pl.when(lv % 2 == 1)
    def _():
        d_sct(0).wait()
        d_sct(1).wait()

    plsc.subcore_barrier()

    def _ochunk(j, carry):
        zc = sid + j * NS

        @pl.when(zc < N_ZCHUNKS)
        def _():
            pltpu.sync_copy(
                acc.at[pl.ds(zc * ZROWS, ZROWS)],
                out_hbm.at[cid, pl.ds(zc * ZROWS, ZROWS)],
            )

        return carry

    lax.fori_loop(0, ZCHUNKS_PER_TILE, _ochunk, 0)


_BN = 1000


def _combine_body(p_ref, nn_ref, o_ref):
    o_ref[...] = (p_ref[0] + p_ref[1]) * nn_ref[...]


def _combine(parts, n_norm2d):
    return pl.pallas_call(
        _combine_body,
        grid=(N_NODES // _BN,),
        in_specs=[
            pl.BlockSpec((NC, _BN, C), lambda i: (0, i, 0)),
            pl.BlockSpec((_BN, 1), lambda i: (i, 0)),
        ],
        out_specs=pl.BlockSpec((_BN, C), lambda i: (i, 0)),
        out_shape=jax.ShapeDtypeStruct((N_NODES, C), jnp.float32),
    )(*parts, n_norm2d)


def kernel(features, radial_basis_function_coefficients, Ys, n_norm, W_R, W_Y,
           map_ab_p_to_a, map_ab_p_to_b):
    ia = map_ab_p_to_a.astype(jnp.int32)
    ib = map_ab_p_to_b.astype(jnp.int32)
    RT = radial_basis_function_coefficients.T
    YT = Ys.T
    kers = [_edge_weights(RT, YT, W_R, W_Y, h) for h in range(N_PHASE)]
    parts = []
    for h in range(N_PHASE):
        sl = slice(h * E_HALF, (h + 1) * E_HALF)
        parts.append(_sc_gather_scatter(features, kers[h], ia[sl], ib[sl]))
    return _combine(parts, n_norm[:, None])


# speedup vs baseline: 1.0109x; 1.0109x over previous
import functools

import jax
import jax.numpy as jnp
from jax import lax
from jax.experimental import pallas as pl
from jax.experimental.pallas import tpu as pltpu
from jax.experimental.pallas import tpu_sc as plsc

N_NODES = 10000
C = 128
N_EDGES = 320000

NC = 2
NS = 16
NW = NC * NS

K_CHUNK = 80
N_PHASE = 2
E_HALF = N_EDGES // N_PHASE
N_CHUNKS = E_HALF // K_CHUNK
CHUNKS_PER_W = -(-N_CHUNKS // NW)

ZROWS = 80
N_ZCHUNKS = N_NODES // ZROWS
ZCHUNKS_PER_TILE = -(-N_ZCHUNKS // NS)


_BE = 16000


_DN = (((0,), (0,)), ((), ()))


def _edge_weights_body(rt_ref, yt_ref, wr_ref, wy_ref, o_ref):
    kr = lax.dot_general(rt_ref[...], wr_ref[...], _DN,
                         preferred_element_type=jnp.float32)
    ky = lax.dot_general(yt_ref[...], wy_ref[...], _DN,
                         preferred_element_type=jnp.float32)
    o_ref[...] = kr * ky


def _edge_weights(RT, YT, W_R, W_Y, half):
    n_radial = RT.shape[0]
    n_sh = YT.shape[0]
    hb = E_HALF // _BE
    return pl.pallas_call(
        _edge_weights_body,
        grid=(hb,),
        in_specs=[
            pl.BlockSpec((n_radial, _BE), lambda i: (0, half * hb + i)),
            pl.BlockSpec((n_sh, _BE), lambda i: (0, half * hb + i)),
            pl.BlockSpec((n_radial, C), lambda i: (0, 0)),
            pl.BlockSpec((n_sh, C), lambda i: (0, 0)),
        ],
        out_specs=pl.BlockSpec((_BE, C), lambda i: (i, 0)),
        out_shape=jax.ShapeDtypeStruct((E_HALF, C), jnp.float32),
    )(RT, YT, W_R, W_Y)


_sc_mesh = plsc.VectorSubcoreMesh(
    core_axis_name="c", subcore_axis_name="s", num_cores=NC, num_subcores=NS
)


@functools.partial(
    pl.kernel,
    out_type=jax.ShapeDtypeStruct((NC, N_NODES, C), jnp.float32),
    mesh=_sc_mesh,
    scratch_types=[
        pltpu.VMEM((2, K_CHUNK), jnp.int32),
        pltpu.VMEM((2, K_CHUNK), jnp.int32),
        pltpu.VMEM((2, K_CHUNK), jnp.int32),
        pltpu.VMEM((2, K_CHUNK, C), jnp.float32),
        pltpu.VMEM((2, K_CHUNK, C), jnp.float32),
        pltpu.VMEM_SHARED((N_NODES, C), jnp.float32),
        pltpu.SemaphoreType.DMA,
        pltpu.SemaphoreType.DMA,
        pltpu.SemaphoreType.DMA,
        pltpu.SemaphoreType.DMA,
        pltpu.SemaphoreType.DMA,
        pltpu.SemaphoreType.DMA,
        pltpu.SemaphoreType.DMA,
        pltpu.SemaphoreType.DMA,
    ],
)
def _sc_gather_scatter(feat_hbm, ker_hbm, ia_hbm, ib_hbm, out_hbm,
                       ib_v, ia_v, ia_sc, feat_v, ker_v, acc,
                       si0, si1, sg0, sg1, sk0, sk1, ss0, ss1):
    cid = lax.axis_index("c")
    sid = lax.axis_index("s")
    wid = cid * NS + sid

    s_idx = (si0, si1)
    s_gat = (sg0, sg1)
    s_ker = (sk0, sk1)
    s_sct = (ss0, ss1)

    def d_ib(b, chunk):
        return pltpu.make_async_copy(
            ib_hbm.at[pl.ds(chunk * K_CHUNK, K_CHUNK)], ib_v.at[b], s_idx[b])

    def d_ia(b, chunk):
        return pltpu.make_async_copy(
            ia_hbm.at[pl.ds(chunk * K_CHUNK, K_CHUNK)], ia_v.at[b], s_idx[b])

    def d_gat(b):
        return pltpu.make_async_copy(feat_hbm.at[ib_v.at[b]], feat_v.at[b],
                                     s_gat[b])

    def d_ker(b, chunk):
        return pltpu.make_async_copy(
            ker_hbm.at[pl.ds(chunk * K_CHUNK, K_CHUNK)], ker_v.at[b], s_ker[b])

    def d_sct(b):
        return pltpu.make_async_copy(ker_v.at[b], acc.at[ia_sc.at[b]],
                                     s_sct[b])

    zero16 = jnp.zeros((16,), jnp.float32)

    def _zrow(r, carry):
        for c8 in range(C // 16):
            feat_v[0, r, pl.ds(c8 * 16, 16)] = zero16
        return carry

    lax.fori_loop(0, ZROWS, _zrow, 0)

    def _zchunk(j, carry):
        zc = sid + j * NS

        @pl.when(zc < N_ZCHUNKS)
        def _():
            pltpu.sync_copy(
                feat_v.at[0, pl.ds(0, ZROWS)],
                acc.at[pl.ds(zc * ZROWS, ZROWS)],
            )

        return carry

    lax.fori_loop(0, ZCHUNKS_PER_TILE, _zchunk, 0)
    plsc.subcore_barrier()

    c0 = wid
    d_ib(0, c0).start()
    d_ia(0, c0).start()
    d_ib(0, c0).wait()
    d_ia(0, c0).wait()
    d_gat(0).start()
    d_ker(0, c0).start()

    @pl.when(c0 + NW < N_CHUNKS)
    def _():
        d_ib(1, c0 + NW).start()
        d_ia(1, c0 + NW).start()

    def _step(i, s):
        o = 1 - s
        ci = wid + i * NW

        @pl.when(ci < N_CHUNKS)
        def _():
            cn = ci + NW
            cn2 = cn + NW
            d_gat(s).wait()
            d_ker(s, ci).wait()
            for g in range(K_CHUNK // 16):
                ia_sc[s, pl.ds(g * 16, 16)] = ia_v[s, pl.ds(g * 16, 16)]

            @pl.when(cn < N_CHUNKS)
            def _():
                d_ib(o, cn).wait()
                d_ia(o, cn).wait()

                @pl.when(i >= 1)
                def _():
                    d_sct(o).wait()

                d_gat(o).start()
                d_ker(o, cn).start()

            @pl.when(cn2 < N_CHUNKS)
            def _():
                d_ib(s, cn2).start()
                d_ia(s, cn2).start()

            def _row(r):
                for c8 in range(C // 16):
                    sl = pl.ds(c8 * 16, 16)
                    ker_v[s, r, sl] = ker_v[s, r, sl] * feat_v[s, r, sl]

            plsc.parallel_loop(0, K_CHUNK, unroll=4)(_row)
            d_sct(s).start(add=True)

    def _pair(it, carry):
        _step(it * 2, 0)
        _step(it * 2 + 1, 1)
        return carry

    lax.fori_loop(0, (CHUNKS_PER_W + 1) // 2, _pair, 0)

    lv = (N_CHUNKS - 1 - wid) // NW

    @pl.when(lv % 2 == 0)
    def _():
        @pl.when(lv >= 1)
        def _():
            d_sct(1).wait()

        d_sct(0).wait()

    @pl.when(lv % 2 == 1)
    def _():
        d_sct(0).wait()
        d_sct(1).wait()

    plsc.subcore_barrier()

    def _ochunk(j, carry):
        zc = sid + j * NS

        @pl.when(zc < N_ZCHUNKS)
        def _():
            pltpu.sync_copy(
                acc.at[pl.ds(zc * ZROWS, ZROWS)],
                out_hbm.at[cid, pl.ds(zc * ZROWS, ZROWS)],
            )

        return carry

    lax.fori_loop(0, ZCHUNKS_PER_TILE, _ochunk, 0)


_BN = 1000


def _combine_body(p_ref, q_ref, nn_ref, o_ref):
    o_ref[...] = (p_ref[0] + p_ref[1] + q_ref[0] + q_ref[1]) * nn_ref[...]


def _combine(parts, n_norm2d):
    return pl.pallas_call(
        _combine_body,
        grid=(N_NODES // _BN,),
        in_specs=[
            pl.BlockSpec((NC, _BN, C), lambda i: (0, i, 0)),
            pl.BlockSpec((NC, _BN, C), lambda i: (0, i, 0)),
            pl.BlockSpec((_BN, 1), lambda i: (i, 0)),
        ],
        out_specs=pl.BlockSpec((_BN, C), lambda i: (i, 0)),
        out_shape=jax.ShapeDtypeStruct((N_NODES, C), jnp.float32),
    )(*parts, n_norm2d)


def kernel(features, radial_basis_function_coefficients, Ys, n_norm, W_R, W_Y,
           map_ab_p_to_a, map_ab_p_to_b):
    ia = map_ab_p_to_a.astype(jnp.int32)
    ib = map_ab_p_to_b.astype(jnp.int32)
    RT = radial_basis_function_coefficients.T
    YT = Ys.T
    kers = [_edge_weights(RT, YT, W_R, W_Y, h) for h in range(N_PHASE)]
    parts = []
    for h in range(N_PHASE):
        sl = slice(h * E_HALF, (h + 1) * E_HALF)
        parts.append(_sc_gather_scatter(features, kers[h], ia[sl], ib[sl]))
    return _combine(parts, n_norm[:, None])
